# Initial kernel scaffold; baseline (speedup 1.0000x reference)
import jax
import jax.numpy as jnp
from jax.experimental import pallas as pl


def kernel(input, input_timestamp, edge_index, emb, W1, b1, W2, b2, pos_emb, Wq, Wk, Wv, Wo, Wf1, bf1, Wf2, bf2, Wout, bout):
    raise NotImplementedError("write your pallas kernel here")


# speedup vs baseline: 5.3502x; 5.3502x over previous
import functools
import math

import jax
import jax.numpy as jnp
from jax import lax
from jax.experimental import pallas as pl
from jax.experimental.pallas import tpu as pltpu
from jax.experimental.pallas import tpu_sc as plsc

N = 10000
NINP = 128
E = 160000
B = 16
SL = 200
L = SL - 1
POS = 8
DM = NINP + POS
NH = 8
DK = DM // NH
NC = 2
NS = 16
NEG = -1e9

EPW_H = E // (NC * NS)
CH_H = 40
EPS = E // NS
CH = 128
NFULL = EPS // CH
REM = EPS - NFULL * CH

BPAD = 3328
RPW = BPAD // (NC * NS)

@functools.cache
def _mesh():
    return plsc.VectorSubcoreMesh(core_axis_name="c", subcore_axis_name="s")


def _hist_body(dst_hbm, ones_hbm, out_hbm, idx_v, ones_v, z_v, acc_sh):
    c = lax.axis_index("c")
    s = lax.axis_index("s")
    pltpu.sync_copy(ones_hbm, ones_v)
    for j in range(640 // 16):
        z_v[pl.ds(j * 16, 16)] = jnp.zeros((16,), jnp.float32)

    @pl.when(s < 15)
    def _():
        pltpu.sync_copy(z_v, acc_sh.at[pl.ds(s * 640, 640)])

    @pl.when(s == 15)
    def _():
        pltpu.sync_copy(z_v.at[pl.ds(0, 400)], acc_sh.at[pl.ds(9600, 400)])

    plsc.subcore_barrier()
    base = (s * NC + c) * EPW_H

    def chunk(i, carry):
        pltpu.sync_copy(dst_hbm.at[pl.ds(base + i * CH_H, CH_H)], idx_v)
        pltpu.sync_copy(ones_v, acc_sh.at[idx_v], add=True)
        return carry

    lax.fori_loop(0, EPW_H // CH_H, chunk, 0)
    plsc.subcore_barrier()

    @pl.when(s < 15)
    def _():
        pltpu.sync_copy(acc_sh.at[pl.ds(s * 640, 640)], z_v)
        pltpu.sync_copy(z_v, out_hbm.at[pl.ds(c * N + s * 640, 640)])

    @pl.when(s == 15)
    def _():
        pltpu.sync_copy(acc_sh.at[pl.ds(9600, 400)], z_v.at[pl.ds(0, 400)])
        pltpu.sync_copy(z_v.at[pl.ds(0, 400)], out_hbm.at[pl.ds(c * N + 9600, 400)])


@functools.cache
def _hist():
    return pl.kernel(
        _hist_body,
        out_type=jax.ShapeDtypeStruct((NC * N,), jnp.float32),
        scratch_types=[
            pltpu.VMEM((CH_H,), jnp.int32),
            pltpu.VMEM((CH_H,), jnp.float32),
            pltpu.VMEM((640,), jnp.float32),
            pltpu.VMEM_SHARED((N,), jnp.float32),
        ],
        mesh=_mesh(),
    )


FS = 128
HN = N // NC
GROWS = 8
ACCR = HN + GROWS


def _filter_dst(idx_ref, nbase, count):
    for t in range(count // 16):
        d = idx_ref[pl.ds(t * 16, 16)]
        dloc = d - nbase
        valid = (dloc >= 0) & (dloc < HN)
        idx_ref[pl.ds(t * 16, 16)] = jnp.where(valid, dloc, HN + (d & 7))


def _make_scatter(npass):
    def body(xws_hbm, srcoff_hbm, dst_hbm, out_hbm,
             idx_s, idx_d, rows, idx_s16, idx_d16, rows16, buf, acc_sh, sem):
        c = lax.axis_index("c")
        s = lax.axis_index("s")
        nbase = c * HN

        for j in range(npass):
            @pl.when(s < 15)
            def _():
                pltpu.sync_copy(xws_hbm.at[pl.ds(j * N + nbase + s * 320, 320)],
                                buf)
                pltpu.sync_copy(buf, acc_sh.at[pl.ds(s * 320, 320)])

            @pl.when(s == 15)
            def _():
                pltpu.sync_copy(xws_hbm.at[pl.ds(j * N + nbase + 4800, 200)],
                                buf.at[pl.ds(0, 200)])
                pltpu.sync_copy(buf.at[pl.ds(0, 200)], acc_sh.at[pl.ds(4800, 200)])

            plsc.subcore_barrier()
            ebase = s * EPS

            def chunk(i, carry):
                off = ebase + i * CH
                pltpu.sync_copy(srcoff_hbm.at[pl.ds(j * E + off, CH)], idx_s)
                pltpu.sync_copy(dst_hbm.at[pl.ds(off, CH)], idx_d)
                _filter_dst(idx_d, nbase, CH)
                pltpu.async_copy(xws_hbm.at[idx_s], rows, sem).wait()
                pltpu.sync_copy(rows, acc_sh.at[idx_d], add=True)
                return carry

            lax.fori_loop(0, NFULL, chunk, 0)

            off = ebase + NFULL * CH
            pltpu.sync_copy(srcoff_hbm.at[pl.ds(j * E + off, REM)], idx_s16)
            pltpu.sync_copy(dst_hbm.at[pl.ds(off, REM)], idx_d16)
            _filter_dst(idx_d16, nbase, REM)
            pltpu.async_copy(xws_hbm.at[idx_s16], rows16, sem).wait()
            pltpu.sync_copy(rows16, acc_sh.at[idx_d16], add=True)

            plsc.subcore_barrier()

            @pl.when(s < 15)
            def _():
                pltpu.sync_copy(acc_sh.at[pl.ds(s * 320, 320)], buf)
                pltpu.sync_copy(buf, out_hbm.at[pl.ds(j * N + nbase + s * 320, 320)])

            @pl.when(s == 15)
            def _():
                pltpu.sync_copy(acc_sh.at[pl.ds(4800, 200)], buf.at[pl.ds(0, 200)])
                pltpu.sync_copy(buf.at[pl.ds(0, 200)],
                                out_hbm.at[pl.ds(j * N + nbase + 4800, 200)])

            if j + 1 < npass:
                plsc.subcore_barrier()

    return pl.kernel(
        body,
        out_type=jax.ShapeDtypeStruct((npass * N, FS), jnp.float32),
        scratch_types=[
            pltpu.VMEM((CH,), jnp.int32),
            pltpu.VMEM((CH,), jnp.int32),
            pltpu.VMEM((CH, FS), jnp.float32),
            pltpu.VMEM((REM,), jnp.int32),
            pltpu.VMEM((REM,), jnp.int32),
            pltpu.VMEM((REM, FS), jnp.float32),
            pltpu.VMEM((320, FS), jnp.float32),
            pltpu.VMEM_SHARED((ACCR, FS), jnp.float32),
            pltpu.SemaphoreType.DMA,
        ],
        mesh=_mesh(),
    )


_scatter_l1 = functools.cache(lambda: _make_scatter(2))
_scatter_l2 = functools.cache(lambda: _make_scatter(1))


def _gather_body(tab_hbm, idx_hbm, out_hbm, idx_v, rows_v, sem):
    c = lax.axis_index("c")
    s = lax.axis_index("s")
    base = (s * NC + c) * RPW
    pltpu.sync_copy(idx_hbm.at[pl.ds(base, RPW)], idx_v)
    pltpu.async_copy(tab_hbm.at[idx_v], rows_v, sem).wait()
    pltpu.sync_copy(rows_v, out_hbm.at[pl.ds(base, RPW)])


@functools.cache
def _gather():
    return pl.kernel(
        _gather_body,
        out_type=jax.ShapeDtypeStruct((BPAD, NINP), jnp.float32),
        scratch_types=[
            pltpu.VMEM((RPW,), jnp.int32),
            pltpu.VMEM((RPW, NINP), jnp.float32),
            pltpu.SemaphoreType.DMA,
        ],
        mesh=_mesh(),
    )


def _kr_body(h_ref, o_ref):
    o_ref[...] = lax.rsqrt(1.0 + h_ref[0:1, :] + h_ref[1:2, :])


def _mm1_body(emb_ref, w_ref, dinv_ref, o_ref):
    o_ref[0] = jnp.dot(emb_ref[...], w_ref[0],
                       preferred_element_type=jnp.float32) * dinv_ref[...]


def _mm2_body(acc_ref, dinv_ref, b1_ref, w2_ref, o_ref):
    dinv = dinv_ref[...]
    g0 = acc_ref[0] * dinv + b1_ref[:, 0:NINP]
    g1 = acc_ref[1] * dinv + b1_ref[:, NINP:2 * NINP]
    o_ref[...] = (jnp.dot(g0, w2_ref[0:NINP, :], preferred_element_type=jnp.float32)
                  + jnp.dot(g1, w2_ref[NINP:2 * NINP, :],
                            preferred_element_type=jnp.float32)) * dinv


def _kne_body(a_ref, dinv_ref, b2_ref, o_ref):
    o_ref[...] = a_ref[...] * dinv_ref[...] + b2_ref[...]


def _t1_body(dy_ref, pos_ref, seq_ref, wq_ref, wk_ref, wv_ref, wo_ref,
             wf1_ref, bf1_ref, wf2_ref, bf2_ref, o_ref):
    x = jnp.concatenate([dy_ref[0], pos_ref[...]], axis=1)
    q = jnp.dot(x, wq_ref[...], preferred_element_type=jnp.float32)
    k = jnp.dot(x, wk_ref[...], preferred_element_type=jnp.float32)
    v = jnp.dot(x, wv_ref[...], preferred_element_type=jnp.float32)
    pad = seq_ref[0] == 0
    rows = lax.broadcasted_iota(jnp.int32, (L, L), 0)
    cols = lax.broadcasted_iota(jnp.int32, (L, L), 1)
    causal = cols <= rows
    padm = jnp.broadcast_to(pad, (L, L))
    lane = lax.broadcasted_iota(jnp.int32, (1, DM), 1)
    scale = jnp.float32(1.0 / math.sqrt(DK))
    ctx = jnp.zeros((L, DM), jnp.float32)
    for h in range(NH):
        hm = ((lane >= h * DK) & (lane < (h + 1) * DK)).astype(jnp.float32)
        sc = lax.dot_general(q * hm, k * hm, (((1,), (1,)), ((), ())),
                             preferred_element_type=jnp.float32) * scale
        sc = jnp.where(causal, sc, NEG)
        sc = jnp.where(padm, NEG, sc)
        m = jnp.max(sc, axis=1, keepdims=True)
        e = jnp.exp(sc - m)
        a = e / jnp.sum(e, axis=1, keepdims=True)
        ctx = ctx + jnp.dot(a, v * hm, preferred_element_type=jnp.float32)
    att0 = jnp.dot(ctx, wo_ref[...], preferred_element_type=jnp.float32)
    ff = jnp.maximum(
        jnp.dot(att0, wf1_ref[...], preferred_element_type=jnp.float32)
        + bf1_ref[...], 0.0)
    ff = jnp.dot(ff, wf2_ref[...], preferred_element_type=jnp.float32) + bf2_ref[...]
    o_ref[0] = att0 + ff


def _t2_body(att_ref, seq_ref, wout_ref, bout_ref, o_ref, mask_ref):
    x = att_ref[0]
    o_ref[0] = jnp.dot(x, wout_ref[...],
                       preferred_element_type=jnp.float32) + bout_ref[...]
    lane = lax.broadcasted_iota(jnp.int32, (1, N), 1)
    mask_ref[...] = jnp.where(lane == 0, -jnp.inf, 0.0).astype(jnp.float32)

    def body(t, carry):
        tok = seq_ref[0, 0, t]
        m = jnp.where(lane == tok, -jnp.inf, mask_ref[...])
        mask_ref[...] = m
        o_ref[0, pl.ds(t, 1), :] = o_ref[0, pl.ds(t, 1), :] + m
        return carry

    lax.fori_loop(0, L, body, 0)


def kernel(input, input_timestamp, edge_index, emb, W1, b1, W2, b2, pos_emb,
           Wq, Wk, Wv, Wo, Wf1, bf1, Wf2, bf2, Wout, bout):
    del input_timestamp
    seq = input[:, :-1]
    src = edge_index[0]
    dst = edge_index[1]
    srcoff2 = jnp.concatenate([src, src + N])
    ones_h = jnp.ones((CH_H,), jnp.float32)

    hist2 = _hist()(dst, ones_h).reshape(NC, N)
    dinv_row = pl.pallas_call(
        _kr_body,
        out_shape=jax.ShapeDtypeStruct((1, N), jnp.float32),
    )(hist2)
    dinv_col = dinv_row.reshape(N, 1)

    xws1 = pl.pallas_call(
        _mm1_body,
        grid=(2,),
        in_specs=[
            pl.BlockSpec((N, NINP), lambda h: (0, 0)),
            pl.BlockSpec((1, NINP, NINP), lambda h: (h, 0, 0)),
            pl.BlockSpec((N, 1), lambda h: (0, 0)),
        ],
        out_specs=pl.BlockSpec((1, N, NINP), lambda h: (h, 0, 0)),
        out_shape=jax.ShapeDtypeStruct((2, N, NINP), jnp.float32),
    )(emb, W1.reshape(NINP, 2, NINP).transpose(1, 0, 2), dinv_col)

    acc1 = _scatter_l1()(xws1.reshape(2 * N, NINP), srcoff2, dst)

    xws2 = pl.pallas_call(
        _mm2_body,
        out_shape=jax.ShapeDtypeStruct((N, NINP), jnp.float32),
    )(acc1.reshape(2, N, NINP), dinv_col, b1.reshape(1, -1), W2)

    acc2 = _scatter_l2()(xws2, src, dst)

    node_emb = pl.pallas_call(
        _kne_body,
        out_shape=jax.ShapeDtypeStruct((N, NINP), jnp.float32),
    )(acc2, dinv_col, b2.reshape(1, -1))

    seqf = seq.reshape(-1)
    seqp = jnp.concatenate(
        [seqf, jnp.zeros((BPAD - B * L,), jnp.int32)])
    dyemb = _gather()(node_emb, seqp)[:B * L].reshape(B, L, NINP)

    seq3 = seq.reshape(B, 1, L)
    att = pl.pallas_call(
        _t1_body,
        grid=(B,),
        in_specs=[
            pl.BlockSpec((1, L, NINP), lambda b: (b, 0, 0)),
            pl.BlockSpec((L, POS), lambda b: (0, 0)),
            pl.BlockSpec((1, 1, L), lambda b: (b, 0, 0)),
            pl.BlockSpec((DM, DM), lambda b: (0, 0)),
            pl.BlockSpec((DM, DM), lambda b: (0, 0)),
            pl.BlockSpec((DM, DM), lambda b: (0, 0)),
            pl.BlockSpec((DM, DM), lambda b: (0, 0)),
            pl.BlockSpec((DM, DM), lambda b: (0, 0)),
            pl.BlockSpec((1, DM), lambda b: (0, 0)),
            pl.BlockSpec((DM, DM), lambda b: (0, 0)),
            pl.BlockSpec((1, DM), lambda b: (0, 0)),
        ],
        out_specs=pl.BlockSpec((1, L, DM), lambda b: (b, 0, 0)),
        out_shape=jax.ShapeDtypeStruct((B, L, DM), jnp.float32),
    )(dyemb, pos_emb[:L], seq3, Wq, Wk, Wv, Wo, Wf1,
      bf1.reshape(1, -1), Wf2, bf2.reshape(1, -1))

    out = pl.pallas_call(
        _t2_body,
        grid=(B,),
        in_specs=[
            pl.BlockSpec((1, L, DM), lambda b: (b, 0, 0)),
            pl.BlockSpec((1, 1, L), lambda b: (b, 0, 0), memory_space=pltpu.SMEM),
            pl.BlockSpec((DM, N), lambda b: (0, 0)),
            pl.BlockSpec((1, N), lambda b: (0, 0)),
        ],
        out_specs=pl.BlockSpec((1, L, N), lambda b: (b, 0, 0)),
        out_shape=jax.ShapeDtypeStruct((B, L, N), jnp.float32),
        scratch_shapes=[pltpu.VMEM((1, N), jnp.float32)],
    )(att, seq3, Wout, bout.reshape(1, -1))

    return out.reshape(-1, N)
